# Initial kernel scaffold; baseline (speedup 1.0000x reference)
import jax
import jax.numpy as jnp
from jax.experimental import pallas as pl


def kernel(x1, x2, x3, emb, emb_dur, emb_vel, W_ih, W_hh, b_ih, b_hh, W_dec, b_dec, W_dur, b_dur, W_vel, b_vel):
    raise NotImplementedError("write your pallas kernel here")


# speedup vs baseline: 2.5345x; 2.5345x over previous
import functools

import jax
import jax.numpy as jnp
from jax import lax
from jax.experimental import pallas as pl
from jax.experimental.pallas import tpu as pltpu
from jax.experimental.pallas import tpu_sc as plsc

_NC = 2
_NS = 16
_NW = _NC * _NS
_CH = 128


def _sc_gather3(xt1, xt2, xt3, emb, emb_dur, emb_vel):
    n_rows_128, _ = xt1.shape
    n = n_rows_128 * _CH
    e = emb.shape[1]
    per_w = n // _NW
    n_ch = per_w // _CH

    mesh = plsc.VectorSubcoreMesh(
        core_axis_name="c", subcore_axis_name="s", num_cores=_NC,
        num_subcores=_NS)

    @functools.partial(
        pl.kernel,
        out_type=[jax.ShapeDtypeStruct((n, e), jnp.float32)] * 3,
        mesh=mesh,
        scratch_types=[pltpu.VMEM((n_ch, _CH), jnp.int32)] * 3
        + [pltpu.VMEM((per_w, e), jnp.float32)] * 3
        + [pltpu.SemaphoreType.DMA] * 3,
        compiler_params=pltpu.CompilerParams(use_tc_tiling_on_sc=False),
    )
    def k(x1h, x2h, x3h, t1h, t2h, t3h, o1h, o2h, o3h,
          i1, i2, i3, r1, r2, r3, s1, s2, s3):
        wid = lax.axis_index("s") * _NC + lax.axis_index("c")
        base = wid * per_w
        pltpu.sync_copy(x1h.at[pl.ds(wid * n_ch, n_ch)], i1)
        pltpu.sync_copy(x2h.at[pl.ds(wid * n_ch, n_ch)], i2)
        pltpu.sync_copy(x3h.at[pl.ds(wid * n_ch, n_ch)], i3)
        copies = []
        for c in range(n_ch):
            dst = pl.ds(c * _CH, _CH)
            copies.append(pltpu.async_copy(t1h.at[i1.at[c]], r1.at[dst], s1))
            copies.append(pltpu.async_copy(t2h.at[i2.at[c]], r2.at[dst], s2))
            copies.append(pltpu.async_copy(t3h.at[i3.at[c]], r3.at[dst], s3))
        for cp in copies:
            cp.wait()
        pltpu.sync_copy(r1, o1h.at[pl.ds(base, per_w)])
        pltpu.sync_copy(r2, o2h.at[pl.ds(base, per_w)])
        pltpu.sync_copy(r3, o3h.at[pl.ds(base, per_w)])

    return k(xt1, xt2, xt3, emb, emb_dur, emb_vel)


def _xproj_body(g1, g2, g3, w, b, o):
    x = g1[...] + g2[...] + g3[...]
    o[...] = jnp.dot(x, w[...], preferred_element_type=jnp.float32) + b[...]


def _xproj(g1, g2, g3, wih_t, bias2d, rb=1024):
    n, e = g1.shape
    h4 = wih_t.shape[1]
    return pl.pallas_call(
        _xproj_body,
        grid=(n // rb,),
        in_specs=[pl.BlockSpec((rb, e), lambda i: (i, 0))] * 3
        + [pl.BlockSpec((e, h4), lambda i: (0, 0)),
           pl.BlockSpec((1, h4), lambda i: (0, 0))],
        out_specs=pl.BlockSpec((rb, h4), lambda i: (i, 0)),
        out_shape=jax.ShapeDtypeStruct((n, h4), jnp.float32),
    )(g1, g2, g3, wih_t, bias2d)


def _sigmoid(x):
    return 0.5 * jnp.tanh(0.5 * x) + 0.5


def _scan_body(xw_ref, whh_ref, ht_ref, cn_ref, h_sc, c_sc, *, hh, ss):
    s = pl.program_id(0)

    @pl.when(s == 0)
    def _():
        h_sc[...] = jnp.zeros_like(h_sc)
        c_sc[...] = jnp.zeros_like(c_sc)

    h = h_sc[...]
    gates = xw_ref[0] + jnp.dot(h, whh_ref[...],
                                preferred_element_type=jnp.float32)
    i = _sigmoid(gates[:, :hh])
    f = _sigmoid(gates[:, hh:2 * hh])
    g = jnp.tanh(gates[:, 2 * hh:3 * hh])
    o = _sigmoid(gates[:, 3 * hh:])
    c = f * c_sc[...] + i * g
    h = o * jnp.tanh(c)
    c_sc[...] = c
    h_sc[...] = h
    ht_ref[...] = h[None]

    @pl.when(s == ss - 1)
    def _():
        cn_ref[...] = c[None]


def _lstm_scan(xw3, whh_t):
    s, b, h4 = xw3.shape
    hh = h4 // 4
    return pl.pallas_call(
        functools.partial(_scan_body, hh=hh, ss=s),
        grid=(s,),
        in_specs=[pl.BlockSpec((1, b, h4), lambda i: (i, 0, 0)),
                  pl.BlockSpec((hh, h4), lambda i: (0, 0))],
        out_specs=[pl.BlockSpec((1, b, hh), lambda i: (i, 0, 0)),
                   pl.BlockSpec((1, b, hh), lambda i: (0, 0, 0))],
        out_shape=[jax.ShapeDtypeStruct((s, b, hh), jnp.float32),
                   jax.ShapeDtypeStruct((1, b, hh), jnp.float32)],
        scratch_shapes=[pltpu.VMEM((b, hh), jnp.float32),
                        pltpu.VMEM((b, hh), jnp.float32)],
    )(xw3, whh_t)


def _log_softmax(z):
    m = jnp.max(z, axis=-1, keepdims=True)
    e = jnp.exp(z - m)
    return z - m - jnp.log(jnp.sum(e, axis=-1, keepdims=True))


def _dec_body(ht_ref, w_ref, b_ref, o1_ref, o2_ref, o3_ref, *, v, dv):
    z = jnp.dot(ht_ref[...], w_ref[...],
                preferred_element_type=jnp.float32) + b_ref[...]
    o1_ref[...] = _log_softmax(z[:, :v])
    o2_ref[...] = _log_softmax(z[:, v:v + dv])
    o3_ref[...] = _log_softmax(z[:, v + dv:])


def _decode(ht2, wcat_t, bcat2d, v, dv, vv, rd=2048):
    n, h = ht2.shape
    w = v + dv + vv
    return pl.pallas_call(
        functools.partial(_dec_body, v=v, dv=dv),
        grid=(n // rd,),
        in_specs=[pl.BlockSpec((rd, h), lambda i: (i, 0)),
                  pl.BlockSpec((h, w), lambda i: (0, 0)),
                  pl.BlockSpec((1, w), lambda i: (0, 0))],
        out_specs=[pl.BlockSpec((rd, v), lambda i: (i, 0)),
                   pl.BlockSpec((rd, dv), lambda i: (i, 0)),
                   pl.BlockSpec((rd, vv), lambda i: (i, 0))],
        out_shape=[jax.ShapeDtypeStruct((n, v), jnp.float32),
                   jax.ShapeDtypeStruct((n, dv), jnp.float32),
                   jax.ShapeDtypeStruct((n, vv), jnp.float32)],
    )(ht2, wcat_t, bcat2d)


def kernel(x1, x2, x3, emb, emb_dur, emb_vel, W_ih, W_hh, b_ih, b_hh,
           W_dec, b_dec, W_dur, b_dur, W_vel, b_vel):
    b, s = x1.shape
    e = emb.shape[1]
    h = W_hh.shape[1]
    v, dv, vv = W_dec.shape[0], W_dur.shape[0], W_vel.shape[0]
    n = b * s

    xt1 = x1.T.reshape(n // _CH, _CH).astype(jnp.int32)
    xt2 = x2.T.reshape(n // _CH, _CH).astype(jnp.int32)
    xt3 = x3.T.reshape(n // _CH, _CH).astype(jnp.int32)

    g1, g2, g3 = _sc_gather3(xt1, xt2, xt3, emb, emb_dur, emb_vel)

    wih_t = W_ih.T
    bias2d = (b_ih + b_hh).reshape(1, -1)
    xw = _xproj(g1, g2, g3, wih_t, bias2d)
    xw3 = xw.reshape(s, b, 4 * h)

    ht3, cn = _lstm_scan(xw3, W_hh.T)
    hn = ht3[s - 1][None]

    wcat_t = jnp.concatenate([W_dec, W_dur, W_vel], axis=0).T
    bcat2d = jnp.concatenate([b_dec, b_dur, b_vel]).reshape(1, -1)
    o1, o2, o3 = _decode(ht3.reshape(n, h), wcat_t, bcat2d, v, dv, vv)

    out = o1.reshape(s, b, v).swapaxes(0, 1)
    out_dur = o2.reshape(s, b, dv).swapaxes(0, 1)
    out_vel = o3.reshape(s, b, vv).swapaxes(0, 1)
    return (out, out_dur, out_vel, (hn, cn))


# speedup vs baseline: 2.7899x; 1.1008x over previous
import functools

import jax
import jax.numpy as jnp
from jax import lax
from jax.experimental import pallas as pl
from jax.experimental.pallas import tpu as pltpu
from jax.experimental.pallas import tpu_sc as plsc

_NC = 2
_NS = 16
_NW = _NC * _NS
_CH = 128


def _sc_gather3(xt1, xt2, xt3, emb, emb_dur, emb_vel):
    n_rows_128, _ = xt1.shape
    n = n_rows_128 * _CH
    e = emb.shape[1]
    per_w = n // _NW
    n_ch = per_w // _CH

    mesh = plsc.VectorSubcoreMesh(
        core_axis_name="c", subcore_axis_name="s", num_cores=_NC,
        num_subcores=_NS)

    @functools.partial(
        pl.kernel,
        out_type=[jax.ShapeDtypeStruct((n, e), jnp.float32)] * 3,
        mesh=mesh,
        scratch_types=[pltpu.VMEM((n_ch, _CH), jnp.int32)] * 3
        + [pltpu.VMEM((per_w, e), jnp.float32)] * 3
        + [pltpu.SemaphoreType.DMA] * 3,
        compiler_params=pltpu.CompilerParams(use_tc_tiling_on_sc=False),
    )
    def k(x1h, x2h, x3h, t1h, t2h, t3h, o1h, o2h, o3h,
          i1, i2, i3, r1, r2, r3, s1, s2, s3):
        wid = lax.axis_index("s") * _NC + lax.axis_index("c")
        base = wid * per_w
        pltpu.sync_copy(x1h.at[pl.ds(wid * n_ch, n_ch)], i1)
        pltpu.sync_copy(x2h.at[pl.ds(wid * n_ch, n_ch)], i2)
        pltpu.sync_copy(x3h.at[pl.ds(wid * n_ch, n_ch)], i3)
        copies = []
        for c in range(n_ch):
            dst = pl.ds(c * _CH, _CH)
            copies.append(pltpu.async_copy(t1h.at[i1.at[c]], r1.at[dst], s1))
            copies.append(pltpu.async_copy(t2h.at[i2.at[c]], r2.at[dst], s2))
            copies.append(pltpu.async_copy(t3h.at[i3.at[c]], r3.at[dst], s3))
        for cp in copies:
            cp.wait()
        pltpu.sync_copy(r1, o1h.at[pl.ds(base, per_w)])
        pltpu.sync_copy(r2, o2h.at[pl.ds(base, per_w)])
        pltpu.sync_copy(r3, o3h.at[pl.ds(base, per_w)])

    return k(xt1, xt2, xt3, emb, emb_dur, emb_vel)


def _xproj_body(g1, g2, g3, w, b, o):
    x = (g1[...] + g2[...] + g3[...]).astype(jnp.bfloat16)
    o[...] = (jnp.dot(x, w[...], preferred_element_type=jnp.float32)
              + b[...]).astype(jnp.bfloat16)


def _xproj(g1, g2, g3, wih_t, bias2d, rb=1024):
    n, e = g1.shape
    h4 = wih_t.shape[1]
    return pl.pallas_call(
        _xproj_body,
        grid=(n // rb,),
        in_specs=[pl.BlockSpec((rb, e), lambda i: (i, 0))] * 3
        + [pl.BlockSpec((e, h4), lambda i: (0, 0)),
           pl.BlockSpec((1, h4), lambda i: (0, 0))],
        out_specs=pl.BlockSpec((rb, h4), lambda i: (i, 0)),
        out_shape=jax.ShapeDtypeStruct((n, h4), jnp.bfloat16),
    )(g1, g2, g3, wih_t, bias2d)


def _sigmoid(x):
    return 0.5 * jnp.tanh(0.5 * x) + 0.5


def _scan_body(xw_ref, whh_ref, ht_ref, hn_ref, cn_ref, h_sc, c_sc, *, hh, ss):
    s = pl.program_id(0)

    @pl.when(s == 0)
    def _():
        h_sc[...] = jnp.zeros_like(h_sc)
        c_sc[...] = jnp.zeros_like(c_sc)

    h = h_sc[...]
    gates = xw_ref[0].astype(jnp.float32) + jnp.dot(
        h.astype(jnp.bfloat16), whh_ref[...],
        preferred_element_type=jnp.float32)
    i = _sigmoid(gates[:, :hh])
    f = _sigmoid(gates[:, hh:2 * hh])
    g = jnp.tanh(gates[:, 2 * hh:3 * hh])
    o = _sigmoid(gates[:, 3 * hh:])
    c = f * c_sc[...] + i * g
    h = o * jnp.tanh(c)
    c_sc[...] = c
    h_sc[...] = h
    ht_ref[...] = h.astype(jnp.bfloat16)[None]

    @pl.when(s == ss - 1)
    def _():
        hn_ref[...] = h[None]
        cn_ref[...] = c[None]


def _lstm_scan(xw3, whh_t):
    s, b, h4 = xw3.shape
    hh = h4 // 4
    return pl.pallas_call(
        functools.partial(_scan_body, hh=hh, ss=s),
        grid=(s,),
        in_specs=[pl.BlockSpec((1, b, h4), lambda i: (i, 0, 0)),
                  pl.BlockSpec((hh, h4), lambda i: (0, 0))],
        out_specs=[pl.BlockSpec((1, b, hh), lambda i: (i, 0, 0)),
                   pl.BlockSpec((1, b, hh), lambda i: (0, 0, 0)),
                   pl.BlockSpec((1, b, hh), lambda i: (0, 0, 0))],
        out_shape=[jax.ShapeDtypeStruct((s, b, hh), jnp.bfloat16),
                   jax.ShapeDtypeStruct((1, b, hh), jnp.float32),
                   jax.ShapeDtypeStruct((1, b, hh), jnp.float32)],
        scratch_shapes=[pltpu.VMEM((b, hh), jnp.float32),
                        pltpu.VMEM((b, hh), jnp.float32)],
    )(xw3, whh_t)


def _log_softmax(z):
    m = jnp.max(z, axis=-1, keepdims=True)
    e = jnp.exp(z - m)
    return z - m - jnp.log(jnp.sum(e, axis=-1, keepdims=True))


def _dec_body(ht_ref, w_ref, b_ref, o1_ref, o2_ref, o3_ref, *, v, dv):
    z = jnp.dot(ht_ref[...], w_ref[...],
                preferred_element_type=jnp.float32) + b_ref[...]
    o1_ref[...] = _log_softmax(z[:, :v])
    o2_ref[...] = _log_softmax(z[:, v:v + dv])
    o3_ref[...] = _log_softmax(z[:, v + dv:])


def _decode(ht2, wcat_t, bcat2d, v, dv, vv, rd=2048):
    n, h = ht2.shape
    w = v + dv + vv
    return pl.pallas_call(
        functools.partial(_dec_body, v=v, dv=dv),
        grid=(n // rd,),
        in_specs=[pl.BlockSpec((rd, h), lambda i: (i, 0)),
                  pl.BlockSpec((h, w), lambda i: (0, 0)),
                  pl.BlockSpec((1, w), lambda i: (0, 0))],
        out_specs=[pl.BlockSpec((rd, v), lambda i: (i, 0)),
                   pl.BlockSpec((rd, dv), lambda i: (i, 0)),
                   pl.BlockSpec((rd, vv), lambda i: (i, 0))],
        out_shape=[jax.ShapeDtypeStruct((n, v), jnp.float32),
                   jax.ShapeDtypeStruct((n, dv), jnp.float32),
                   jax.ShapeDtypeStruct((n, vv), jnp.float32)],
    )(ht2, wcat_t, bcat2d)


def kernel(x1, x2, x3, emb, emb_dur, emb_vel, W_ih, W_hh, b_ih, b_hh,
           W_dec, b_dec, W_dur, b_dur, W_vel, b_vel):
    b, s = x1.shape
    e = emb.shape[1]
    h = W_hh.shape[1]
    v, dv, vv = W_dec.shape[0], W_dur.shape[0], W_vel.shape[0]
    n = b * s

    xt1 = x1.T.reshape(n // _CH, _CH).astype(jnp.int32)
    xt2 = x2.T.reshape(n // _CH, _CH).astype(jnp.int32)
    xt3 = x3.T.reshape(n // _CH, _CH).astype(jnp.int32)

    g1, g2, g3 = _sc_gather3(xt1, xt2, xt3, emb, emb_dur, emb_vel)

    wih_t = W_ih.T.astype(jnp.bfloat16)
    bias2d = (b_ih + b_hh).reshape(1, -1)
    xw = _xproj(g1, g2, g3, wih_t, bias2d)
    xw3 = xw.reshape(s, b, 4 * h)

    ht3, hn, cn = _lstm_scan(xw3, W_hh.T.astype(jnp.bfloat16))

    wcat_t = jnp.concatenate(
        [W_dec, W_dur, W_vel], axis=0).T.astype(jnp.bfloat16)
    bcat2d = jnp.concatenate([b_dec, b_dur, b_vel]).reshape(1, -1)
    o1, o2, o3 = _decode(ht3.reshape(n, h), wcat_t, bcat2d, v, dv, vv)

    out = o1.reshape(s, b, v).swapaxes(0, 1)
    out_dur = o2.reshape(s, b, dv).swapaxes(0, 1)
    out_vel = o3.reshape(s, b, vv).swapaxes(0, 1)
    return (out, out_dur, out_vel, (hn, cn))


# speedup vs baseline: 3.6385x; 1.3042x over previous
import functools

import jax
import jax.numpy as jnp
from jax import lax
from jax.experimental import pallas as pl
from jax.experimental.pallas import tpu as pltpu
from jax.experimental.pallas import tpu_sc as plsc

_NC = 2
_NS = 16
_NW = _NC * _NS
_CH = 128


def _sc_gather3(xt1, xt2, xt3, emb, emb_dur, emb_vel):
    n_rows_128, _ = xt1.shape
    n = n_rows_128 * _CH
    e = emb.shape[1]
    per_w = n // _NW
    n_ch = per_w // _CH

    mesh = plsc.VectorSubcoreMesh(
        core_axis_name="c", subcore_axis_name="s", num_cores=_NC,
        num_subcores=_NS)

    @functools.partial(
        pl.kernel,
        out_type=[jax.ShapeDtypeStruct((n, e), jnp.float32)] * 3,
        mesh=mesh,
        scratch_types=[pltpu.VMEM((n_ch, _CH), jnp.int32)] * 3
        + [pltpu.VMEM((per_w, e), jnp.float32)] * 3
        + [pltpu.SemaphoreType.DMA] * 3,
        compiler_params=pltpu.CompilerParams(use_tc_tiling_on_sc=False),
    )
    def k(x1h, x2h, x3h, t1h, t2h, t3h, o1h, o2h, o3h,
          i1, i2, i3, r1, r2, r3, s1, s2, s3):
        wid = lax.axis_index("s") * _NC + lax.axis_index("c")
        base = wid * per_w
        pltpu.sync_copy(x1h.at[pl.ds(wid * n_ch, n_ch)], i1)
        pltpu.sync_copy(x2h.at[pl.ds(wid * n_ch, n_ch)], i2)
        pltpu.sync_copy(x3h.at[pl.ds(wid * n_ch, n_ch)], i3)
        copies = []
        for c in range(n_ch):
            dst = pl.ds(c * _CH, _CH)
            copies.append(pltpu.async_copy(t1h.at[i1.at[c]], r1.at[dst], s1))
            copies.append(pltpu.async_copy(t2h.at[i2.at[c]], r2.at[dst], s2))
            copies.append(pltpu.async_copy(t3h.at[i3.at[c]], r3.at[dst], s3))
        for cp in copies:
            cp.wait()
        pltpu.sync_copy(r1, o1h.at[pl.ds(base, per_w)])
        pltpu.sync_copy(r2, o2h.at[pl.ds(base, per_w)])
        pltpu.sync_copy(r3, o3h.at[pl.ds(base, per_w)])

    return k(xt1, xt2, xt3, emb, emb_dur, emb_vel)


def _sigmoid(x):
    return 0.5 * jnp.tanh(0.5 * x) + 0.5


def _scan_body(g1_ref, g2_ref, g3_ref, w_ref, ht_ref, hn_ref, cn_ref,
               h_sc, c_sc, *, hh, nblk, ck):
    sblk = pl.program_id(0)

    @pl.when(sblk == 0)
    def _():
        h_sc[...] = jnp.zeros_like(h_sc)
        c_sc[...] = jnp.zeros_like(c_sc)

    b = h_sc.shape[0]
    x = (g1_ref[...] + g2_ref[...] + g3_ref[...]).astype(jnp.bfloat16)
    ones = jnp.ones((b, 1), jnp.bfloat16)
    h = h_sc[...]
    c = c_sc[...]
    w = w_ref[...]
    for k in range(ck):
        lhs = jnp.concatenate([h.astype(jnp.bfloat16), x[k], ones], axis=1)
        gates = jnp.dot(lhs, w, preferred_element_type=jnp.float32)
        i = _sigmoid(gates[:, :hh])
        f = _sigmoid(gates[:, hh:2 * hh])
        g = jnp.tanh(gates[:, 2 * hh:3 * hh])
        o = _sigmoid(gates[:, 3 * hh:])
        c = f * c + i * g
        h = o * jnp.tanh(c)
        ht_ref[k] = h.astype(jnp.bfloat16)
    c_sc[...] = c
    h_sc[...] = h

    @pl.when(sblk == nblk - 1)
    def _():
        hn_ref[...] = h[None]
        cn_ref[...] = c[None]


def _lstm_scan(g1, g2, g3, w_all, s, b, e, hh, ck=8):
    h4 = 4 * hh
    nblk = s // ck
    ke = w_all.shape[0]
    g1 = g1.reshape(s, b, e)
    g2 = g2.reshape(s, b, e)
    g3 = g3.reshape(s, b, e)
    return pl.pallas_call(
        functools.partial(_scan_body, hh=hh, nblk=nblk, ck=ck),
        grid=(nblk,),
        in_specs=[pl.BlockSpec((ck, b, e), lambda i: (i, 0, 0))] * 3
        + [pl.BlockSpec((ke, h4), lambda i: (0, 0))],
        out_specs=[pl.BlockSpec((ck, b, hh), lambda i: (i, 0, 0)),
                   pl.BlockSpec((1, b, hh), lambda i: (0, 0, 0)),
                   pl.BlockSpec((1, b, hh), lambda i: (0, 0, 0))],
        out_shape=[jax.ShapeDtypeStruct((s, b, hh), jnp.bfloat16),
                   jax.ShapeDtypeStruct((1, b, hh), jnp.float32),
                   jax.ShapeDtypeStruct((1, b, hh), jnp.float32)],
        scratch_shapes=[pltpu.VMEM((b, hh), jnp.float32),
                        pltpu.VMEM((b, hh), jnp.float32)],
    )(g1, g2, g3, w_all)


def _log_softmax(z):
    m = jnp.max(z, axis=-1, keepdims=True)
    e = jnp.exp(z - m)
    return z - m - jnp.log(jnp.sum(e, axis=-1, keepdims=True))


def _dec_body(ht_ref, w_ref, b_ref, o1_ref, o2_ref, o3_ref, *, v, dv):
    z = jnp.dot(ht_ref[...], w_ref[...],
                preferred_element_type=jnp.float32) + b_ref[...]
    o1_ref[...] = _log_softmax(z[:, :v])
    o2_ref[...] = _log_softmax(z[:, v:v + dv])
    o3_ref[...] = _log_softmax(z[:, v + dv:])


def _decode(ht2, wcat_t, bcat2d, v, dv, vv, rd=2048):
    n, h = ht2.shape
    w = v + dv + vv
    return pl.pallas_call(
        functools.partial(_dec_body, v=v, dv=dv),
        grid=(n // rd,),
        in_specs=[pl.BlockSpec((rd, h), lambda i: (i, 0)),
                  pl.BlockSpec((h, w), lambda i: (0, 0)),
                  pl.BlockSpec((1, w), lambda i: (0, 0))],
        out_specs=[pl.BlockSpec((rd, v), lambda i: (i, 0)),
                   pl.BlockSpec((rd, dv), lambda i: (i, 0)),
                   pl.BlockSpec((rd, vv), lambda i: (i, 0))],
        out_shape=[jax.ShapeDtypeStruct((n, v), jnp.float32),
                   jax.ShapeDtypeStruct((n, dv), jnp.float32),
                   jax.ShapeDtypeStruct((n, vv), jnp.float32)],
    )(ht2, wcat_t, bcat2d)


def kernel(x1, x2, x3, emb, emb_dur, emb_vel, W_ih, W_hh, b_ih, b_hh,
           W_dec, b_dec, W_dur, b_dur, W_vel, b_vel):
    b, s = x1.shape
    e = emb.shape[1]
    h = W_hh.shape[1]
    v, dv, vv = W_dec.shape[0], W_dur.shape[0], W_vel.shape[0]
    n = b * s

    xt1 = x1.T.reshape(n // _CH, _CH).astype(jnp.int32)
    xt2 = x2.T.reshape(n // _CH, _CH).astype(jnp.int32)
    xt3 = x3.T.reshape(n // _CH, _CH).astype(jnp.int32)

    g1, g2, g3 = _sc_gather3(xt1, xt2, xt3, emb, emb_dur, emb_vel)

    w_all = jnp.concatenate(
        [W_hh.T, W_ih.T, (b_ih + b_hh).reshape(1, -1)],
        axis=0).astype(jnp.bfloat16)

    ht3, hn, cn = _lstm_scan(g1, g2, g3, w_all, s, b, e, h)

    wcat_t = jnp.concatenate(
        [W_dec, W_dur, W_vel], axis=0).T.astype(jnp.bfloat16)
    bcat2d = jnp.concatenate([b_dec, b_dur, b_vel]).reshape(1, -1)
    o1, o2, o3 = _decode(ht3.reshape(n, h), wcat_t, bcat2d, v, dv, vv)

    out = o1.reshape(s, b, v).swapaxes(0, 1)
    out_dur = o2.reshape(s, b, dv).swapaxes(0, 1)
    out_vel = o3.reshape(s, b, vv).swapaxes(0, 1)
    return (out, out_dur, out_vel, (hn, cn))


# speedup vs baseline: 3.7154x; 1.0211x over previous
import functools

import jax
import jax.numpy as jnp
from jax import lax
from jax.experimental import pallas as pl
from jax.experimental.pallas import tpu as pltpu
from jax.experimental.pallas import tpu_sc as plsc

_NC = 2
_NS = 16
_NW = _NC * _NS
_CH = 128


def _sc_gather3(xt1, xt2, xt3, emb, emb_dur, emb_vel):
    n_rows_128, _ = xt1.shape
    n = n_rows_128 * _CH
    e = emb.shape[1]
    per_w = n // _NW
    n_ch = per_w // _CH

    mesh = plsc.VectorSubcoreMesh(
        core_axis_name="c", subcore_axis_name="s", num_cores=_NC,
        num_subcores=_NS)

    @functools.partial(
        pl.kernel,
        out_type=[jax.ShapeDtypeStruct((n, e), jnp.float32)] * 3,
        mesh=mesh,
        scratch_types=[pltpu.VMEM((n_ch, _CH), jnp.int32)] * 3
        + [pltpu.VMEM((per_w, e), jnp.float32)] * 3
        + [pltpu.SemaphoreType.DMA] * 3,
        compiler_params=pltpu.CompilerParams(use_tc_tiling_on_sc=False),
    )
    def k(x1h, x2h, x3h, t1h, t2h, t3h, o1h, o2h, o3h,
          i1, i2, i3, r1, r2, r3, s1, s2, s3):
        wid = lax.axis_index("s") * _NC + lax.axis_index("c")
        base = wid * per_w
        pltpu.sync_copy(x1h.at[pl.ds(wid * n_ch, n_ch)], i1)
        pltpu.sync_copy(x2h.at[pl.ds(wid * n_ch, n_ch)], i2)
        pltpu.sync_copy(x3h.at[pl.ds(wid * n_ch, n_ch)], i3)
        copies = []
        for c in range(n_ch):
            dst = pl.ds(c * _CH, _CH)
            copies.append(pltpu.async_copy(t1h.at[i1.at[c]], r1.at[dst], s1))
            copies.append(pltpu.async_copy(t2h.at[i2.at[c]], r2.at[dst], s2))
            copies.append(pltpu.async_copy(t3h.at[i3.at[c]], r3.at[dst], s3))
        for cp in copies:
            cp.wait()
        pltpu.sync_copy(r1, o1h.at[pl.ds(base, per_w)])
        pltpu.sync_copy(r2, o2h.at[pl.ds(base, per_w)])
        pltpu.sync_copy(r3, o3h.at[pl.ds(base, per_w)])

    return k(xt1, xt2, xt3, emb, emb_dur, emb_vel)


def _sigmoid(x):
    return 0.5 * jnp.tanh(0.5 * x) + 0.5


def _scan_body(g1_ref, g2_ref, g3_ref, w_ref, ht_ref, hn_ref, cn_ref,
               h_sc, c_sc, *, hh, nblk, ck):
    sblk = pl.program_id(0)

    @pl.when(sblk == 0)
    def _():
        h_sc[...] = jnp.zeros_like(h_sc)
        c_sc[...] = jnp.zeros_like(c_sc)

    b = h_sc.shape[0]
    x = (g1_ref[...] + g2_ref[...] + g3_ref[...]).astype(jnp.bfloat16)
    ones = jnp.ones((b, 1), jnp.bfloat16)
    h = h_sc[...]
    c = c_sc[...]
    w = w_ref[...]
    for k in range(ck):
        lhs = jnp.concatenate([h.astype(jnp.bfloat16), x[k], ones], axis=1)
        gates = jnp.dot(lhs, w, preferred_element_type=jnp.float32)
        i = _sigmoid(gates[:, :hh])
        f = _sigmoid(gates[:, hh:2 * hh])
        g = jnp.tanh(gates[:, 2 * hh:3 * hh])
        o = _sigmoid(gates[:, 3 * hh:])
        c = f * c + i * g
        h = o * jnp.tanh(c)
        ht_ref[k] = h.astype(jnp.bfloat16)
    c_sc[...] = c
    h_sc[...] = h

    @pl.when(sblk == nblk - 1)
    def _():
        hn_ref[...] = h[None]
        cn_ref[...] = c[None]


def _lstm_scan(g1, g2, g3, w_all, s, b, e, hh, ck=16):
    h4 = 4 * hh
    nblk = s // ck
    ke = w_all.shape[0]
    g1 = g1.reshape(s, b, e)
    g2 = g2.reshape(s, b, e)
    g3 = g3.reshape(s, b, e)
    return pl.pallas_call(
        functools.partial(_scan_body, hh=hh, nblk=nblk, ck=ck),
        grid=(nblk,),
        in_specs=[pl.BlockSpec((ck, b, e), lambda i: (i, 0, 0))] * 3
        + [pl.BlockSpec((ke, h4), lambda i: (0, 0))],
        out_specs=[pl.BlockSpec((ck, b, hh), lambda i: (i, 0, 0)),
                   pl.BlockSpec((1, b, hh), lambda i: (0, 0, 0)),
                   pl.BlockSpec((1, b, hh), lambda i: (0, 0, 0))],
        out_shape=[jax.ShapeDtypeStruct((s, b, hh), jnp.bfloat16),
                   jax.ShapeDtypeStruct((1, b, hh), jnp.float32),
                   jax.ShapeDtypeStruct((1, b, hh), jnp.float32)],
        scratch_shapes=[pltpu.VMEM((b, hh), jnp.float32),
                        pltpu.VMEM((b, hh), jnp.float32)],
    )(g1, g2, g3, w_all)


def _log_softmax_nomax(z):
    return z - jnp.log(jnp.sum(jnp.exp(z), axis=-1, keepdims=True))


def _dec_body(ht_ref, w_ref, b_ref, o1_ref, o2_ref, o3_ref, *, v, dv):
    z = jnp.dot(ht_ref[...], w_ref[...],
                preferred_element_type=jnp.float32) + b_ref[...]
    o1_ref[...] = _log_softmax_nomax(z[:, :v])
    o2_ref[...] = _log_softmax_nomax(z[:, v:v + dv])
    o3_ref[...] = _log_softmax_nomax(z[:, v + dv:])


def _decode(ht2, wcat_t, bcat2d, v, dv, vv, rd=2048):
    n, h = ht2.shape
    w = v + dv + vv
    return pl.pallas_call(
        functools.partial(_dec_body, v=v, dv=dv),
        grid=(n // rd,),
        in_specs=[pl.BlockSpec((rd, h), lambda i: (i, 0)),
                  pl.BlockSpec((h, w), lambda i: (0, 0)),
                  pl.BlockSpec((1, w), lambda i: (0, 0))],
        out_specs=[pl.BlockSpec((rd, v), lambda i: (i, 0)),
                   pl.BlockSpec((rd, dv), lambda i: (i, 0)),
                   pl.BlockSpec((rd, vv), lambda i: (i, 0))],
        out_shape=[jax.ShapeDtypeStruct((n, v), jnp.float32),
                   jax.ShapeDtypeStruct((n, dv), jnp.float32),
                   jax.ShapeDtypeStruct((n, vv), jnp.float32)],
    )(ht2, wcat_t, bcat2d)


def kernel(x1, x2, x3, emb, emb_dur, emb_vel, W_ih, W_hh, b_ih, b_hh,
           W_dec, b_dec, W_dur, b_dur, W_vel, b_vel):
    b, s = x1.shape
    e = emb.shape[1]
    h = W_hh.shape[1]
    v, dv, vv = W_dec.shape[0], W_dur.shape[0], W_vel.shape[0]
    n = b * s

    xt1 = x1.T.reshape(n // _CH, _CH).astype(jnp.int32)
    xt2 = x2.T.reshape(n // _CH, _CH).astype(jnp.int32)
    xt3 = x3.T.reshape(n // _CH, _CH).astype(jnp.int32)

    g1, g2, g3 = _sc_gather3(xt1, xt2, xt3, emb, emb_dur, emb_vel)

    w_all = jnp.concatenate(
        [W_hh.T, W_ih.T, (b_ih + b_hh).reshape(1, -1)],
        axis=0).astype(jnp.bfloat16)

    ht3, hn, cn = _lstm_scan(g1, g2, g3, w_all, s, b, e, h)

    wcat_t = jnp.concatenate(
        [W_dec, W_dur, W_vel], axis=0).T.astype(jnp.bfloat16)
    bcat2d = jnp.concatenate([b_dec, b_dur, b_vel]).reshape(1, -1)
    o1, o2, o3 = _decode(ht3.reshape(n, h), wcat_t, bcat2d, v, dv, vv)

    out = o1.reshape(s, b, v).swapaxes(0, 1)
    out_dur = o2.reshape(s, b, dv).swapaxes(0, 1)
    out_vel = o3.reshape(s, b, vv).swapaxes(0, 1)
    return (out, out_dur, out_vel, (hn, cn))


# speedup vs baseline: 4.0162x; 1.0810x over previous
import functools

import jax
import jax.numpy as jnp
from jax import lax
from jax.experimental import pallas as pl
from jax.experimental.pallas import tpu as pltpu
from jax.experimental.pallas import tpu_sc as plsc

_NC = 2
_NS = 16
_NW = _NC * _NS
_CH = 128


def _sc_gather3(xt1, xt2, xt3, emb, emb_dur, emb_vel):
    n_rows_128, _ = xt1.shape
    n = n_rows_128 * _CH
    e = emb.shape[1]
    per_w = n // _NW
    n_ch = per_w // _CH

    mesh = plsc.VectorSubcoreMesh(
        core_axis_name="c", subcore_axis_name="s", num_cores=_NC,
        num_subcores=_NS)

    @functools.partial(
        pl.kernel,
        out_type=[jax.ShapeDtypeStruct((n, e), jnp.float32)] * 3,
        mesh=mesh,
        scratch_types=[pltpu.VMEM((n_ch, _CH), jnp.int32)] * 3
        + [pltpu.VMEM((per_w, e), jnp.float32)] * 3
        + [pltpu.SemaphoreType.DMA] * 3,
        compiler_params=pltpu.CompilerParams(use_tc_tiling_on_sc=False),
    )
    def k(x1h, x2h, x3h, t1h, t2h, t3h, o1h, o2h, o3h,
          i1, i2, i3, r1, r2, r3, s1, s2, s3):
        wid = lax.axis_index("s") * _NC + lax.axis_index("c")
        base = wid * per_w
        pltpu.sync_copy(x1h.at[pl.ds(wid * n_ch, n_ch)], i1)
        pltpu.sync_copy(x2h.at[pl.ds(wid * n_ch, n_ch)], i2)
        pltpu.sync_copy(x3h.at[pl.ds(wid * n_ch, n_ch)], i3)
        copies = []
        for c in range(n_ch):
            dst = pl.ds(c * _CH, _CH)
            copies.append(pltpu.async_copy(t1h.at[i1.at[c]], r1.at[dst], s1))
            copies.append(pltpu.async_copy(t2h.at[i2.at[c]], r2.at[dst], s2))
            copies.append(pltpu.async_copy(t3h.at[i3.at[c]], r3.at[dst], s3))
        for cp in copies:
            cp.wait()
        pltpu.sync_copy(r1, o1h.at[pl.ds(base, per_w)])
        pltpu.sync_copy(r2, o2h.at[pl.ds(base, per_w)])
        pltpu.sync_copy(r3, o3h.at[pl.ds(base, per_w)])

    return k(xt1, xt2, xt3, emb, emb_dur, emb_vel)


def _sigmoid(x):
    return 0.5 * jnp.tanh(0.5 * x) + 0.5


def _scan_body(g1_ref, g2_ref, g3_ref, w_ref, ht_ref, hn_ref, cn_ref,
               h_sc, c_sc, *, hh, nblk, ck):
    sblk = pl.program_id(0)

    @pl.when(sblk == 0)
    def _():
        h_sc[...] = jnp.zeros_like(h_sc)
        c_sc[...] = jnp.zeros_like(c_sc)

    b = h_sc.shape[0]
    x = (g1_ref[...] + g2_ref[...] + g3_ref[...]).astype(jnp.bfloat16)
    ones = jnp.ones((b, 1), jnp.bfloat16)
    h = h_sc[...]
    c = c_sc[...]
    w = w_ref[...]
    for k in range(ck):
        lhs = jnp.concatenate([h.astype(jnp.bfloat16), x[k], ones], axis=1)
        gates = jnp.dot(lhs, w, preferred_element_type=jnp.float32)
        i = _sigmoid(gates[:, :hh])
        f = _sigmoid(gates[:, hh:2 * hh])
        g = jnp.tanh(gates[:, 2 * hh:3 * hh])
        o = _sigmoid(gates[:, 3 * hh:])
        c = f * c + i * g
        h = o * jnp.tanh(c)
        ht_ref[k] = h.astype(jnp.bfloat16)
    c_sc[...] = c
    h_sc[...] = h

    @pl.when(sblk == nblk - 1)
    def _():
        hn_ref[...] = h[None]
        cn_ref[...] = c[None]


def _lstm_scan(g1, g2, g3, w_all, s, b, e, hh, ck=16):
    h4 = 4 * hh
    nblk = s // ck
    ke = w_all.shape[0]
    g1 = g1.reshape(s, b, e)
    g2 = g2.reshape(s, b, e)
    g3 = g3.reshape(s, b, e)
    return pl.pallas_call(
        functools.partial(_scan_body, hh=hh, nblk=nblk, ck=ck),
        grid=(nblk,),
        in_specs=[pl.BlockSpec((ck, b, e), lambda i: (i, 0, 0))] * 3
        + [pl.BlockSpec((ke, h4), lambda i: (0, 0))],
        out_specs=[pl.BlockSpec((ck, b, hh), lambda i: (i, 0, 0)),
                   pl.BlockSpec((1, b, hh), lambda i: (0, 0, 0)),
                   pl.BlockSpec((1, b, hh), lambda i: (0, 0, 0))],
        out_shape=[jax.ShapeDtypeStruct((s, b, hh), jnp.bfloat16),
                   jax.ShapeDtypeStruct((1, b, hh), jnp.float32),
                   jax.ShapeDtypeStruct((1, b, hh), jnp.float32)],
        scratch_shapes=[pltpu.VMEM((b, hh), jnp.float32),
                        pltpu.VMEM((b, hh), jnp.float32)],
    )(g1, g2, g3, w_all)


def _log_softmax_nomax(z):
    return z - jnp.log(jnp.sum(jnp.exp(z), axis=-1, keepdims=True))


def _dec_body(ht_ref, w_ref, b_ref, o1_ref, o2_ref, o3_ref, *, v, dv, b, sc):
    z = jnp.dot(ht_ref[...], w_ref[...],
                preferred_element_type=jnp.float32) + b_ref[...]

    def put(o_ref, seg):
        o_ref[...] = _log_softmax_nomax(seg).reshape(
            sc, b, seg.shape[1]).swapaxes(0, 1)

    put(o1_ref, z[:, :v])
    put(o2_ref, z[:, v:v + dv])
    put(o3_ref, z[:, v + dv:])


def _decode(ht2, wcat_t, bcat2d, v, dv, vv, s, b, rd=2048):
    n, h = ht2.shape
    w = v + dv + vv
    sc = rd // b
    return pl.pallas_call(
        functools.partial(_dec_body, v=v, dv=dv, b=b, sc=sc),
        grid=(n // rd,),
        in_specs=[pl.BlockSpec((rd, h), lambda i: (i, 0)),
                  pl.BlockSpec((h, w), lambda i: (0, 0)),
                  pl.BlockSpec((1, w), lambda i: (0, 0))],
        out_specs=[pl.BlockSpec((b, sc, v), lambda i: (0, i, 0)),
                   pl.BlockSpec((b, sc, dv), lambda i: (0, i, 0)),
                   pl.BlockSpec((b, sc, vv), lambda i: (0, i, 0))],
        out_shape=[jax.ShapeDtypeStruct((b, s, v), jnp.float32),
                   jax.ShapeDtypeStruct((b, s, dv), jnp.float32),
                   jax.ShapeDtypeStruct((b, s, vv), jnp.float32)],
    )(ht2, wcat_t, bcat2d)


def kernel(x1, x2, x3, emb, emb_dur, emb_vel, W_ih, W_hh, b_ih, b_hh,
           W_dec, b_dec, W_dur, b_dur, W_vel, b_vel):
    b, s = x1.shape
    e = emb.shape[1]
    h = W_hh.shape[1]
    v, dv, vv = W_dec.shape[0], W_dur.shape[0], W_vel.shape[0]
    n = b * s

    xt1 = x1.T.reshape(n // _CH, _CH).astype(jnp.int32)
    xt2 = x2.T.reshape(n // _CH, _CH).astype(jnp.int32)
    xt3 = x3.T.reshape(n // _CH, _CH).astype(jnp.int32)

    g1, g2, g3 = _sc_gather3(xt1, xt2, xt3, emb, emb_dur, emb_vel)

    w_all = jnp.concatenate(
        [W_hh.T, W_ih.T, (b_ih + b_hh).reshape(1, -1)],
        axis=0).astype(jnp.bfloat16)

    ht3, hn, cn = _lstm_scan(g1, g2, g3, w_all, s, b, e, h)

    wcat_t = jnp.concatenate(
        [W_dec, W_dur, W_vel], axis=0).T.astype(jnp.bfloat16)
    bcat2d = jnp.concatenate([b_dec, b_dur, b_vel]).reshape(1, -1)
    out, out_dur, out_vel = _decode(
        ht3.reshape(n, h), wcat_t, bcat2d, v, dv, vv, s, b)
    return (out, out_dur, out_vel, (hn, cn))
